# Initial kernel scaffold; baseline (speedup 1.0000x reference)
import jax
import jax.numpy as jnp
from jax.experimental import pallas as pl


def kernel(x, edge_index, W1, b1, W2m, b2m, W2s, b2s, Wm, bm, Ws, bs):
    raise NotImplementedError("write your pallas kernel here")


# speedup vs baseline: 8.3194x; 8.3194x over previous
import functools

import jax
import jax.numpy as jnp
from jax import lax
from jax.experimental import pallas as pl
from jax.experimental.pallas import tpu as pltpu
from jax.experimental.pallas import tpu_sc as plsc

N = 10000
E = 320000
ALPHA = 0.1
D = 128

N_TILES = 16
ROWS_PER_TILE = 640
N_PAD = N_TILES * ROWS_PER_TILE
DUMMY = N
CHUNK = 128
CHUNKS_PER_TILE = 79
E_PAD = 2 * N_TILES * CHUNKS_PER_TILE * CHUNK
ROW_CHUNKS = ROWS_PER_TILE // CHUNK

_MESH = plsc.VectorSubcoreMesh(core_axis_name="c", subcore_axis_name="s")


DEG_CHUNKS = 157
E_PAD_DEG = N_TILES * DEG_CHUNKS * CHUNK


@functools.partial(
    pl.kernel,
    out_type=jax.ShapeDtypeStruct((2, N_PAD, 16), jnp.float32),
    mesh=_MESH,
    scratch_types=[
        pltpu.VMEM((CHUNK,), jnp.int32),
        pltpu.VMEM((CHUNK,), jnp.int32),
        pltpu.VMEM((CHUNK, 16), jnp.float32),
        pltpu.VMEM((CHUNK, 16), jnp.float32),
        pltpu.VMEM_SHARED((N_PAD, 16), jnp.float32),
    ],
    compiler_params=pltpu.CompilerParams(use_tc_tiling_on_sc=False),
)
def _sc_degrees(ei_flat, iota_flat, ones_h, out, idx_c, iot_c, ones_v, zbuf_v,
                deg_sh):
    c = lax.axis_index("c")
    s = lax.axis_index("s")
    pltpu.sync_copy(ones_h.at[0], ones_v)
    pltpu.sync_copy(ones_h.at[1], zbuf_v)

    row0 = s * ROWS_PER_TILE
    for k in range(ROW_CHUNKS):
        pltpu.sync_copy(iota_flat.at[pl.ds(row0 + k * CHUNK, CHUNK)], iot_c)
        pltpu.sync_copy(zbuf_v, deg_sh.at[iot_c])
    plsc.subcore_barrier()

    base = (c * N_TILES + s) * (DEG_CHUNKS * CHUNK)

    def body(j, _):
        pltpu.sync_copy(ei_flat.at[pl.ds(base + j * CHUNK, CHUNK)], idx_c)
        pltpu.sync_copy(ones_v, deg_sh.at[idx_c], add=True)
        return 0
    lax.fori_loop(0, DEG_CHUNKS, body, 0)
    plsc.subcore_barrier()

    for k in range(ROW_CHUNKS):
        pltpu.sync_copy(iota_flat.at[pl.ds(row0 + k * CHUNK, CHUNK)], iot_c)
        pltpu.sync_copy(deg_sh.at[iot_c], zbuf_v)
        pltpu.sync_copy(zbuf_v, out.at[c, pl.ds(row0 + k * CHUNK, CHUNK)])


@functools.partial(
    pl.kernel,
    out_type=jax.ShapeDtypeStruct((2, N_PAD, D), jnp.float32),
    mesh=_MESH,
    scratch_types=[
        pltpu.VMEM((CHUNK,), jnp.int32),
        pltpu.VMEM((CHUNK,), jnp.int32),
        pltpu.VMEM((CHUNK,), jnp.int32),
        pltpu.VMEM((CHUNK, D), jnp.float32),
        pltpu.VMEM((CHUNK, D), jnp.float32),
        pltpu.VMEM_SHARED((N_PAD, D), jnp.float32),
    ],
    compiler_params=pltpu.CompilerParams(use_tc_tiling_on_sc=False),
)
def _sc_edge_pass(w_hbm, src_flat, dst_flat, iota_flat, zeros_h, out,
                  idx_s, idx_d, iot_c, rows_v, zrow_v, acc_sh):
    c = lax.axis_index("c")
    s = lax.axis_index("s")
    pltpu.sync_copy(zeros_h, zrow_v)

    row0 = s * ROWS_PER_TILE
    for k in range(ROW_CHUNKS):
        pltpu.sync_copy(iota_flat.at[pl.ds(row0 + k * CHUNK, CHUNK)], iot_c)
        pltpu.sync_copy(zrow_v, acc_sh.at[iot_c])
    plsc.subcore_barrier()

    base = (c * N_TILES + s) * (CHUNKS_PER_TILE * CHUNK)

    def body(j, _):
        pltpu.sync_copy(src_flat.at[pl.ds(base + j * CHUNK, CHUNK)], idx_s)
        pltpu.sync_copy(dst_flat.at[pl.ds(base + j * CHUNK, CHUNK)], idx_d)
        pltpu.sync_copy(w_hbm.at[idx_s], rows_v)
        pltpu.sync_copy(rows_v, acc_sh.at[idx_d], add=True)
        return 0
    lax.fori_loop(0, CHUNKS_PER_TILE, body, 0)
    plsc.subcore_barrier()

    for k in range(ROW_CHUNKS):
        pltpu.sync_copy(iota_flat.at[pl.ds(row0 + k * CHUNK, CHUNK)], iot_c)
        pltpu.sync_copy(acc_sh.at[iot_c], rows_v)
        pltpu.sync_copy(rows_v, out.at[c, pl.ds(row0 + k * CHUNK, CHUNK)])


_TC_GRID = 8
_TC_BLK = N_PAD // _TC_GRID


def _tc_init_body(x_ref, wc_ref, g0_ref, dego_ref, out_ref):
    dof = lax.rsqrt(jnp.maximum(dego_ref[...], 1.0))
    h = jnp.dot(x_ref[...], wc_ref[...],
                preferred_element_type=jnp.float32) + g0_ref[...]
    out_ref[...] = dof * h


def _tc_init(x_pad, wc, g0, deg_out):
    return pl.pallas_call(
        _tc_init_body,
        grid=(_TC_GRID,),
        in_specs=[
            pl.BlockSpec((_TC_BLK, D), lambda i: (i, 0)),
            pl.BlockSpec((D, D), lambda i: (0, 0)),
            pl.BlockSpec((1, D), lambda i: (0, 0)),
            pl.BlockSpec((_TC_BLK, 1), lambda i: (i, 0)),
        ],
        out_specs=pl.BlockSpec((_TC_BLK, D), lambda i: (i, 0)),
        out_shape=jax.ShapeDtypeStruct((N_PAD, D), jnp.float32),
    )(x_pad, wc, g0, deg_out)


def _tc_combine_body(w_ref, acc_ref, g_ref, dego_ref, degi_ref, out_ref, *,
                     final):
    dego = jnp.maximum(dego_ref[...], 1.0)
    dof = lax.rsqrt(dego)
    q = (1.0 - ALPHA) * dof * lax.rsqrt(jnp.maximum(degi_ref[...], 1.0))
    agg = acc_ref[0] + acc_ref[1]
    n = ALPHA * w_ref[...] + q * agg + dof * g_ref[...]
    if final:
        n = jnp.sqrt(dego) * n
    out_ref[...] = n


def _tc_combine(w, acc, g, deg_out, deg_in, final):
    return pl.pallas_call(
        functools.partial(_tc_combine_body, final=final),
        grid=(_TC_GRID,),
        in_specs=[
            pl.BlockSpec((_TC_BLK, D), lambda i: (i, 0)),
            pl.BlockSpec((2, _TC_BLK, D), lambda i: (0, i, 0)),
            pl.BlockSpec((1, D), lambda i: (0, 0)),
            pl.BlockSpec((_TC_BLK, 1), lambda i: (i, 0)),
            pl.BlockSpec((_TC_BLK, 1), lambda i: (i, 0)),
        ],
        out_specs=pl.BlockSpec((_TC_BLK, D), lambda i: (i, 0)),
        out_shape=jax.ShapeDtypeStruct((N_PAD, D), jnp.float32),
    )(w, acc, g, deg_out, deg_in)


def kernel(x, edge_index, W1, b1, W2m, b2m, W2s, b2s, Wm, bm, Ws, bs):
    x = x.astype(jnp.float32)

    wc = jnp.concatenate([W1 @ W2m @ Wm, W1 @ W2s @ Ws], axis=1)
    g0 = jnp.concatenate([b1 @ W2m @ Wm, b1 @ W2s @ Ws])[None, :]
    g1 = jnp.concatenate([b2m @ Wm, b2s @ Ws])[None, :]
    g2 = jnp.concatenate([bm, bs])[None, :]
    gz = jnp.zeros((1, D), jnp.float32)

    src = edge_index[0]
    dst = edge_index[1]

    padd = E_PAD_DEG - E
    srcd = jnp.concatenate([src, jnp.full((padd,), DUMMY, jnp.int32)])
    dstd = jnp.concatenate([dst, jnp.full((padd,), DUMMY, jnp.int32)])
    ei_flat = jnp.concatenate([dstd, srcd])

    pade = E_PAD - E
    src_flat = jnp.concatenate([src, jnp.full((pade,), DUMMY, jnp.int32)])
    dst_flat = jnp.concatenate([dst, jnp.full((pade,), DUMMY, jnp.int32)])

    iota_flat = jnp.arange(N_PAD, dtype=jnp.int32)
    ones_h = jnp.stack([jnp.ones((CHUNK, 16), jnp.float32),
                        jnp.zeros((CHUNK, 16), jnp.float32)])
    zeros_h = jnp.zeros((CHUNK, D), jnp.float32)

    deg = _sc_degrees(ei_flat, iota_flat, ones_h)
    deg_in = deg[0, :, 0:1]
    deg_out = deg[1, :, 0:1]

    x_pad = jnp.pad(x, ((0, N_PAD - N), (0, 0)))
    w = _tc_init(x_pad, wc, g0, deg_out)

    for g in (g1, g2):
        acc = _sc_edge_pass(w, src_flat, dst_flat, iota_flat, zeros_h)
        w = _tc_combine(w, acc, g, deg_out, deg_in, final=False)
    acc = _sc_edge_pass(w, src_flat, dst_flat, iota_flat, zeros_h)
    out = _tc_combine(w, acc, gz, deg_out, deg_in, final=True)

    return out[:N, :64], out[:N, 64:]
